# Initial kernel scaffold; baseline (speedup 1.0000x reference)
import functools

import jax
import jax.numpy as jnp
from jax import lax
from jax.experimental import pallas as pl
from jax.experimental.pallas import tpu as pltpu
from jax.experimental.pallas import tpu_sc as plsc

V = 50000
K = 512
D = 256
N = 32768

NC = 2
NS = 16
NW = NC * NS
TOK_PER_TILE = N // NW
GCHUNK = 128
NCHUNK = TOK_PER_TILE // GCHUNK


def _sc_gather_bincount(doc, emb_w, zeros_v, ones_g):
    mesh = plsc.VectorSubcoreMesh(core_axis_name="c", subcore_axis_name="s")

    @functools.partial(
        pl.kernel,
        mesh=mesh,
        out_type=(
            jax.ShapeDtypeStruct((N, D), jnp.float32),
            jax.ShapeDtypeStruct((NC, V), jnp.float32),
        ),
        scratch_types=[
            pltpu.VMEM((NCHUNK, GCHUNK), jnp.int32),
            pltpu.VMEM((GCHUNK, D), jnp.float32),
            pltpu.VMEM((GCHUNK,), jnp.float32),
            pltpu.VMEM_SHARED((V,), jnp.float32),
            pltpu.SemaphoreType.DMA,
        ],
    )
    def k(doc_hbm, emb_hbm, zeros_hbm, ones_hbm, out_hbm, bc_hbm,
          idx_v, rows_v, ones_v, hist_sh, sem):
        cid = lax.axis_index("c")
        sid = lax.axis_index("s")
        wid = sid * NC + cid
        base = wid * TOK_PER_TILE

        @pl.when(sid == 0)
        def _():
            pltpu.sync_copy(zeros_hbm, hist_sh)

        pltpu.sync_copy(ones_hbm, ones_v)
        for j in range(NCHUNK):
            pltpu.sync_copy(doc_hbm.at[pl.ds(base + j * GCHUNK, GCHUNK)],
                            idx_v.at[j])
        plsc.subcore_barrier()

        for j in range(NCHUNK):
            pltpu.async_copy(emb_hbm.at[idx_v.at[j]], rows_v, sem).wait()
            pltpu.sync_copy(rows_v,
                            out_hbm.at[pl.ds(base + j * GCHUNK, GCHUNK)])
            pltpu.sync_copy(ones_v, hist_sh.at[idx_v.at[j]], add=True)

        plsc.subcore_barrier()

        @pl.when(sid == 0)
        def _():
            pltpu.sync_copy(hist_sh, bc_hbm.at[cid])

    return k(doc, emb_w, zeros_v, ones_g)


BN = 512
NB = N // BN


def _vq_body(e_ref, c_ref, enc_ref, q_ref, docu_ref, vq_ref, acc_ref, vqs_ref):
    i = pl.program_id(0)
    e = e_ref[...]
    c = c_ref[...]
    e2 = jnp.sum(e * e, axis=1, keepdims=True)
    c2 = jnp.sum(c * c, axis=1)
    cross = lax.dot_general(e, c, (((1,), (1,)), ((), ())))
    dist = e2 + c2[None, :] - 2.0 * cross
    m = jnp.min(dist, axis=1, keepdims=True)
    kiota = lax.broadcasted_iota(jnp.int32, (BN, K), 1)
    idx = jnp.min(jnp.where(dist == m, kiota, K), axis=1, keepdims=True)
    onehot = (kiota == idx).astype(jnp.float32)
    q = jnp.dot(onehot, c)
    enc_ref[...] = onehot
    q_ref[...] = q

    @pl.when(i == 0)
    def _():
        acc_ref[...] = jnp.zeros_like(acc_ref)
        vqs_ref[0, 0] = 0.0

    acc_ref[...] += jnp.sum(q, axis=0, keepdims=True)
    diff = q - e
    vqs_ref[0, 0] += jnp.sum(diff * diff)

    @pl.when(i == NB - 1)
    def _():
        docu_ref[...] = acc_ref[...] / N
        mloss = vqs_ref[0, 0] / (N * D)
        vq_ref[0, 0] = mloss + 0.25 * mloss


def _tc_vq(embedded, cw):
    return pl.pallas_call(
        _vq_body,
        grid=(NB,),
        in_specs=[
            pl.BlockSpec((BN, D), lambda i: (i, 0)),
            pl.BlockSpec((K, D), lambda i: (0, 0)),
        ],
        out_specs=[
            pl.BlockSpec((BN, K), lambda i: (i, 0)),
            pl.BlockSpec((BN, D), lambda i: (i, 0)),
            pl.BlockSpec((1, D), lambda i: (0, 0)),
            pl.BlockSpec((1, 1), lambda i: (0, 0)),
        ],
        out_shape=[
            jax.ShapeDtypeStruct((N, K), jnp.float32),
            jax.ShapeDtypeStruct((N, D), jnp.float32),
            jax.ShapeDtypeStruct((1, D), jnp.float32),
            jax.ShapeDtypeStruct((1, 1), jnp.float32),
        ],
        scratch_shapes=[
            pltpu.VMEM((1, D), jnp.float32),
            pltpu.SMEM((1, 1), jnp.float32),
        ],
    )(embedded, cw)


def _lts_body(c_ref, out_ref):
    c = c_ref[...]
    g = lax.dot_general(c, c, (((1,), (1,)), ((), ())))
    nrm = jnp.sum(c * c, axis=1)
    sm = jnp.sum(c, axis=1)
    d2 = (nrm[:, None] + nrm[None, :] - 2.0 * g
          + 2e-6 * (sm[:, None] - sm[None, :]) + D * 1e-12)
    dist = jnp.sqrt(jnp.maximum(d2, 0.0))
    r = lax.broadcasted_iota(jnp.int32, (K, K), 0)
    cc = lax.broadcasted_iota(jnp.int32, (K, K), 1)
    losses = jnp.where(r == cc, dist, jnp.maximum(0.0, 1.0 - dist))
    out_ref[0, 0] = jnp.sum(losses) / (K * K)


def _tc_lts(cw):
    return pl.pallas_call(
        _lts_body,
        out_shape=jax.ShapeDtypeStruct((1, 1), jnp.float32),
    )(cw)


BV = 2048
NVB = (V + BV - 1) // BV


def _logits_body(w_ref, b_ref, docu_ref, lg_ref, m_ref, s_ref, mm_ref, ss_ref):
    j = pl.program_id(0)
    w = w_ref[...]
    docu = docu_ref[...]
    lg = lax.dot_general(docu, w, (((1,), (1,)), ((), ()))) + b_ref[...]
    lg_ref[...] = lg
    viota = lax.broadcasted_iota(jnp.int32, (1, BV), 1) + j * BV
    valid = viota < V
    lgv = jnp.where(valid, lg, -jnp.inf)
    bm = jnp.max(lgv)

    @pl.when(j == 0)
    def _():
        mm_ref[0, 0] = -jnp.inf
        ss_ref[0, 0] = 0.0

    m_old = mm_ref[0, 0]
    m_new = jnp.maximum(m_old, bm)
    ssum = jnp.sum(jnp.where(valid, jnp.exp(lg - m_new), 0.0))
    ss_ref[0, 0] = ss_ref[0, 0] * jnp.exp(m_old - m_new) + ssum
    mm_ref[0, 0] = m_new

    @pl.when(j == NVB - 1)
    def _():
        m_ref[0, 0] = mm_ref[0, 0]
        s_ref[0, 0] = ss_ref[0, 0]


def _tc_logits(q2v_W, q2v_b2d, docu):
    return pl.pallas_call(
        _logits_body,
        grid=(NVB,),
        in_specs=[
            pl.BlockSpec((BV, D), lambda j: (j, 0)),
            pl.BlockSpec((1, BV), lambda j: (0, j)),
            pl.BlockSpec((1, D), lambda j: (0, 0)),
        ],
        out_specs=[
            pl.BlockSpec((1, BV), lambda j: (0, j)),
            pl.BlockSpec((1, 1), lambda j: (0, 0)),
            pl.BlockSpec((1, 1), lambda j: (0, 0)),
        ],
        out_shape=[
            jax.ShapeDtypeStruct((1, V), jnp.float32),
            jax.ShapeDtypeStruct((1, 1), jnp.float32),
            jax.ShapeDtypeStruct((1, 1), jnp.float32),
        ],
        scratch_shapes=[
            pltpu.SMEM((1, 1), jnp.float32),
            pltpu.SMEM((1, 1), jnp.float32),
        ],
    )(q2v_W, q2v_b2d, docu)


def _final_body(lg_ref, m_ref, s_ref, bc_ref, out_ref):
    lg = lg_ref[...]
    smax = jnp.exp(lg - m_ref[0, 0]) / s_ref[0, 0]
    bc = jnp.sum(bc_ref[...], axis=0, keepdims=True)
    out_ref[...] = jnp.log(smax + 1e-6) * bc


def _tc_finalize(lg, m, s, bc2):
    return pl.pallas_call(
        _final_body,
        grid=(NVB,),
        in_specs=[
            pl.BlockSpec((1, BV), lambda j: (0, j)),
            pl.BlockSpec((1, 1), lambda j: (0, 0), memory_space=pltpu.SMEM),
            pl.BlockSpec((1, 1), lambda j: (0, 0), memory_space=pltpu.SMEM),
            pl.BlockSpec((NC, BV), lambda j: (0, j)),
        ],
        out_specs=pl.BlockSpec((1, BV), lambda j: (0, j)),
        out_shape=jax.ShapeDtypeStruct((1, V), jnp.float32),
    )(lg, m, s, bc2)


def kernel(input_document, emb_w, emb_concept_w, q2v_W, q2v_b):
    doc = input_document.astype(jnp.int32)
    zeros_v = jnp.zeros((V,), jnp.float32)
    ones_g = jnp.ones((GCHUNK,), jnp.float32)
    embedded, bc2 = _sc_gather_bincount(doc, emb_w, zeros_v, ones_g)
    enc, qw, docu, vq = _tc_vq(embedded, emb_concept_w)
    lts = _tc_lts(emb_concept_w)
    lg, m, s = _tc_logits(q2v_W, q2v_b.reshape(1, V), docu)
    outs = _tc_finalize(lg, m, s, bc2)
    return (enc, qw, docu, outs, vq.reshape(()), lts.reshape(()))


# speedup vs baseline: 2.2145x; 2.2145x over previous
import functools

import jax
import jax.numpy as jnp
from jax import lax
from jax.experimental import pallas as pl
from jax.experimental.pallas import tpu as pltpu
from jax.experimental.pallas import tpu_sc as plsc

V = 50000
K = 512
D = 256
N = 32768

NC = 2
NS = 16
NW = NC * NS
TOK_PER_TILE = N // NW
GCHUNK = 128
NCHUNK = TOK_PER_TILE // GCHUNK


def _sc_gather_bincount(doc, emb_w, zeros_v, ones_g):
    mesh = plsc.VectorSubcoreMesh(core_axis_name="c", subcore_axis_name="s")

    @functools.partial(
        pl.kernel,
        mesh=mesh,
        out_type=(
            jax.ShapeDtypeStruct((N, D), jnp.float32),
            jax.ShapeDtypeStruct((NC, V), jnp.float32),
        ),
        scratch_types=[
            pltpu.VMEM((NCHUNK, GCHUNK), jnp.int32),
            pltpu.VMEM((GCHUNK, D), jnp.float32),
            pltpu.VMEM((GCHUNK,), jnp.float32),
            pltpu.VMEM_SHARED((V,), jnp.float32),
            pltpu.SemaphoreType.DMA,
        ],
    )
    def k(doc_hbm, emb_hbm, zeros_hbm, ones_hbm, out_hbm, bc_hbm,
          idx_v, rows_v, ones_v, hist_sh, sem):
        cid = lax.axis_index("c")
        sid = lax.axis_index("s")
        wid = sid * NC + cid
        base = wid * TOK_PER_TILE

        @pl.when(sid == 0)
        def _():
            pltpu.sync_copy(zeros_hbm, hist_sh)

        pltpu.sync_copy(ones_hbm, ones_v)
        for j in range(NCHUNK):
            pltpu.sync_copy(doc_hbm.at[pl.ds(base + j * GCHUNK, GCHUNK)],
                            idx_v.at[j])
        plsc.subcore_barrier()

        for j in range(NCHUNK):
            pltpu.async_copy(emb_hbm.at[idx_v.at[j]], rows_v, sem).wait()
            pltpu.sync_copy(rows_v,
                            out_hbm.at[pl.ds(base + j * GCHUNK, GCHUNK)])
            pltpu.sync_copy(ones_v, hist_sh.at[idx_v.at[j]], add=True)

        plsc.subcore_barrier()

        @pl.when(sid == 0)
        def _():
            pltpu.sync_copy(hist_sh, bc_hbm.at[cid])

    return k(doc, emb_w, zeros_v, ones_g)


BN = 512
NB = N // BN


def _vq_body(e_ref, c_ref, enc_ref, q_ref, docu_ref, vq_ref, acc_ref, vqs_ref):
    i = pl.program_id(0)
    e = e_ref[...]
    c = c_ref[...]
    e2 = jnp.sum(e * e, axis=1, keepdims=True)
    c2 = jnp.sum(c * c, axis=1)
    cross = lax.dot_general(e, c, (((1,), (1,)), ((), ())))
    dist = e2 + c2[None, :] - 2.0 * cross
    m = jnp.min(dist, axis=1, keepdims=True)
    kiota = lax.broadcasted_iota(jnp.int32, (BN, K), 1)
    idx = jnp.min(jnp.where(dist == m, kiota, K), axis=1, keepdims=True)
    onehot = (kiota == idx).astype(jnp.float32)
    q = jnp.dot(onehot, c)
    enc_ref[...] = onehot
    q_ref[...] = q

    @pl.when(i == 0)
    def _():
        acc_ref[...] = jnp.zeros_like(acc_ref)
        vqs_ref[0, 0] = 0.0

    acc_ref[...] += jnp.sum(q, axis=0, keepdims=True)
    diff = q - e
    vqs_ref[0, 0] += jnp.sum(diff * diff)

    @pl.when(i == NB - 1)
    def _():
        docu_ref[...] = acc_ref[...] / N
        mloss = vqs_ref[0, 0] / (N * D)
        vq_ref[0, 0] = mloss + 0.25 * mloss


def _tc_vq(embedded, cw):
    return pl.pallas_call(
        _vq_body,
        grid=(NB,),
        in_specs=[
            pl.BlockSpec((BN, D), lambda i: (i, 0)),
            pl.BlockSpec((K, D), lambda i: (0, 0)),
        ],
        out_specs=[
            pl.BlockSpec((BN, K), lambda i: (i, 0)),
            pl.BlockSpec((BN, D), lambda i: (i, 0)),
            pl.BlockSpec((1, D), lambda i: (0, 0)),
            pl.BlockSpec((1, 1), lambda i: (0, 0), memory_space=pltpu.SMEM),
        ],
        out_shape=[
            jax.ShapeDtypeStruct((N, K), jnp.float32),
            jax.ShapeDtypeStruct((N, D), jnp.float32),
            jax.ShapeDtypeStruct((1, D), jnp.float32),
            jax.ShapeDtypeStruct((1, 1), jnp.float32),
        ],
        scratch_shapes=[
            pltpu.VMEM((1, D), jnp.float32),
            pltpu.SMEM((1, 1), jnp.float32),
        ],
    )(embedded, cw)


def _lts_body(c_ref, out_ref):
    c = c_ref[...]
    g = lax.dot_general(c, c, (((1,), (1,)), ((), ())))
    nrm = jnp.sum(c * c, axis=1)
    sm = jnp.sum(c, axis=1)
    d2 = (nrm[:, None] + nrm[None, :] - 2.0 * g
          + 2e-6 * (sm[:, None] - sm[None, :]) + D * 1e-12)
    dist = jnp.sqrt(jnp.maximum(d2, 0.0))
    r = lax.broadcasted_iota(jnp.int32, (K, K), 0)
    cc = lax.broadcasted_iota(jnp.int32, (K, K), 1)
    losses = jnp.where(r == cc, dist, jnp.maximum(0.0, 1.0 - dist))
    out_ref[0, 0] = jnp.sum(losses) / (K * K)


def _tc_lts(cw):
    return pl.pallas_call(
        _lts_body,
        out_specs=pl.BlockSpec(memory_space=pltpu.SMEM),
        out_shape=jax.ShapeDtypeStruct((1, 1), jnp.float32),
    )(cw)


BV = 2048
NVB = (V + BV - 1) // BV


def _logits_body(w_ref, b_ref, docu_ref, lg_ref, m_ref, s_ref, mm_ref, ss_ref):
    j = pl.program_id(0)
    w = w_ref[...]
    docu = docu_ref[...]
    lg = lax.dot_general(docu, w, (((1,), (1,)), ((), ()))) + b_ref[...]
    lg_ref[...] = lg
    viota = lax.broadcasted_iota(jnp.int32, (1, BV), 1) + j * BV
    valid = viota < V
    lgv = jnp.where(valid, lg, -jnp.inf)
    bm = jnp.max(lgv)

    @pl.when(j == 0)
    def _():
        mm_ref[0, 0] = -jnp.inf
        ss_ref[0, 0] = 0.0

    m_old = mm_ref[0, 0]
    m_new = jnp.maximum(m_old, bm)
    ssum = jnp.sum(jnp.where(valid, jnp.exp(lg - m_new), 0.0))
    ss_ref[0, 0] = ss_ref[0, 0] * jnp.exp(m_old - m_new) + ssum
    mm_ref[0, 0] = m_new

    @pl.when(j == NVB - 1)
    def _():
        m_ref[0, 0] = mm_ref[0, 0]
        s_ref[0, 0] = ss_ref[0, 0]


def _tc_logits(q2v_W, q2v_b2d, docu):
    return pl.pallas_call(
        _logits_body,
        grid=(NVB,),
        in_specs=[
            pl.BlockSpec((BV, D), lambda j: (j, 0)),
            pl.BlockSpec((1, BV), lambda j: (0, j)),
            pl.BlockSpec((1, D), lambda j: (0, 0)),
        ],
        out_specs=[
            pl.BlockSpec((1, BV), lambda j: (0, j)),
            pl.BlockSpec((1, 1), lambda j: (0, 0), memory_space=pltpu.SMEM),
            pl.BlockSpec((1, 1), lambda j: (0, 0), memory_space=pltpu.SMEM),
        ],
        out_shape=[
            jax.ShapeDtypeStruct((1, V), jnp.float32),
            jax.ShapeDtypeStruct((1, 1), jnp.float32),
            jax.ShapeDtypeStruct((1, 1), jnp.float32),
        ],
        scratch_shapes=[
            pltpu.SMEM((1, 1), jnp.float32),
            pltpu.SMEM((1, 1), jnp.float32),
        ],
    )(q2v_W, q2v_b2d, docu)


def _final_body(lg_ref, m_ref, s_ref, bc_ref, out_ref):
    lg = lg_ref[...]
    smax = jnp.exp(lg - m_ref[0, 0]) / s_ref[0, 0]
    bc = jnp.sum(bc_ref[...], axis=0, keepdims=True)
    out_ref[...] = jnp.log(smax + 1e-6) * bc


def _tc_finalize(lg, m, s, bc2):
    return pl.pallas_call(
        _final_body,
        grid=(NVB,),
        in_specs=[
            pl.BlockSpec((1, BV), lambda j: (0, j)),
            pl.BlockSpec((1, 1), lambda j: (0, 0), memory_space=pltpu.SMEM),
            pl.BlockSpec((1, 1), lambda j: (0, 0), memory_space=pltpu.SMEM),
            pl.BlockSpec((NC, BV), lambda j: (0, j)),
        ],
        out_specs=pl.BlockSpec((1, BV), lambda j: (0, j)),
        out_shape=jax.ShapeDtypeStruct((1, V), jnp.float32),
    )(lg, m, s, bc2)


def kernel(input_document, emb_w, emb_concept_w, q2v_W, q2v_b):
    doc = input_document.astype(jnp.int32)
    zeros_v = jnp.zeros((V,), jnp.float32)
    ones_g = jnp.ones((GCHUNK,), jnp.float32)
    embedded, bc2 = _sc_gather_bincount(doc, emb_w, zeros_v, ones_g)
    enc, qw, docu, vq = _tc_vq(embedded, emb_concept_w)
    lts = _tc_lts(emb_concept_w)
    lg, m, s = _tc_logits(q2v_W, q2v_b.reshape(1, V), docu)
    outs = _tc_finalize(lg, m, s, bc2)
    return (enc, qw, docu, outs, vq.reshape(()), lts.reshape(()))
